# baseline (speedup 1.0000x reference)
import functools

import jax
import jax.numpy as jnp
from jax import lax
from jax.experimental import pallas as pl
from jax.experimental.pallas import tpu as pltpu
from jax.experimental.pallas import tpu_sc as plsc

VOCAB = 1000000
EMB_DIM = 64
BATCH = 16384

NUM_CORES = 2
NUM_SUBCORES = 16
NUM_WORKERS = NUM_CORES * NUM_SUBCORES
B_PER_W = BATCH // NUM_WORKERS
LANES = 16

_mesh = plsc.VectorSubcoreMesh(core_axis_name="c", subcore_axis_name="s")


@functools.partial(
    pl.kernel,
    mesh=_mesh,
    compiler_params=pltpu.CompilerParams(needs_layout_passes=False),
    out_type=jax.ShapeDtypeStruct((BATCH * EMB_DIM,), jnp.float32),
    scratch_types=[
        pltpu.VMEM((B_PER_W,), jnp.int32),
        pltpu.VMEM((B_PER_W * EMB_DIM,), jnp.float32),
        pltpu.SemaphoreType.DMA,
    ],
)
def _sc_gather(idx_hbm, table_hbm, out_hbm, idx_v, rows_v, sem):
    wid = lax.axis_index("s") * NUM_CORES + lax.axis_index("c")
    base = wid * B_PER_W
    lane_ids = lax.broadcasted_iota(jnp.int32, (LANES,), 0)

    pltpu.sync_copy(idx_hbm.at[pl.ds(base, B_PER_W)], idx_v)

    def group(g, _):
        v = idx_v[pl.ds(g * LANES, LANES)]
        for l in range(LANES):
            s = jnp.max(jnp.where(lane_ids == l, v, 0))
            pltpu.async_copy(
                table_hbm.at[pl.ds(s * EMB_DIM, EMB_DIM)],
                rows_v.at[pl.ds((g * LANES + l) * EMB_DIM, EMB_DIM)],
                sem,
            )
        return ()

    lax.fori_loop(0, B_PER_W // LANES, group, (), unroll=False)

    pltpu.make_async_copy(
        table_hbm.at[pl.ds(0, B_PER_W * EMB_DIM)], rows_v, sem).wait()

    pltpu.sync_copy(
        rows_v, out_hbm.at[pl.ds(base * EMB_DIM, B_PER_W * EMB_DIM)])


def kernel(x, table):
    idx = x.astype(jnp.int32)
    flat = _sc_gather(idx, table.reshape(VOCAB * EMB_DIM))
    return flat.reshape(BATCH, EMB_DIM)


# speedup vs baseline: 2.5602x; 2.5602x over previous
import functools

import jax
import jax.numpy as jnp
from jax import lax
from jax.experimental import pallas as pl
from jax.experimental.pallas import tpu as pltpu
from jax.experimental.pallas import tpu_sc as plsc

VOCAB = 1000000
EMB_DIM = 64
BATCH = 16384

NUM_CORES = 2
NUM_SUBCORES = 16
NUM_WORKERS = NUM_CORES * NUM_SUBCORES
B_PER_W = BATCH // NUM_WORKERS
LANES = 16

_mesh = plsc.VectorSubcoreMesh(core_axis_name="c", subcore_axis_name="s")


@functools.partial(
    pl.kernel,
    mesh=_mesh,
    compiler_params=pltpu.CompilerParams(needs_layout_passes=False),
    out_type=jax.ShapeDtypeStruct((BATCH, 1, EMB_DIM), jnp.float32),
    scratch_types=[
        pltpu.VMEM((B_PER_W,), jnp.int32),
        pltpu.VMEM((B_PER_W, 1, EMB_DIM), jnp.float32),
        pltpu.SemaphoreType.DMA,
    ],
)
def _sc_gather(idx_hbm, table_hbm, out_hbm, idx_v, rows_v, sem):
    wid = lax.axis_index("s") * NUM_CORES + lax.axis_index("c")
    base = wid * B_PER_W
    lane_ids = lax.broadcasted_iota(jnp.int32, (LANES,), 0)

    pltpu.sync_copy(idx_hbm.at[pl.ds(base, B_PER_W)], idx_v)

    def group(g, _):
        v = idx_v[pl.ds(g * LANES, LANES)]
        for l in range(LANES):
            s = jnp.max(jnp.where(lane_ids == l, v, 0))
            pltpu.async_copy(
                table_hbm.at[pl.ds(s >> 3, 1), pl.ds(s & 7, 1)],
                rows_v.at[pl.ds(g * LANES + l, 1)],
                sem,
            )
        return ()

    lax.fori_loop(0, B_PER_W // LANES, group, (), unroll=False)

    pltpu.make_async_copy(
        table_hbm.at[pl.ds(0, B_PER_W), pl.ds(0, 1)], rows_v, sem).wait()

    pltpu.sync_copy(rows_v, out_hbm.at[pl.ds(base, B_PER_W)])


def kernel(x, table):
    idx = x.astype(jnp.int32)
    out3 = _sc_gather(idx, table.reshape(VOCAB // 8, 8, EMB_DIM))
    return out3.reshape(BATCH, EMB_DIM)


# speedup vs baseline: 2.5614x; 1.0005x over previous
import functools

import jax
import jax.numpy as jnp
from jax import lax
from jax.experimental import pallas as pl
from jax.experimental.pallas import tpu as pltpu
from jax.experimental.pallas import tpu_sc as plsc

VOCAB = 1000000
EMB_DIM = 64
BATCH = 16384

NUM_CORES = 2
NUM_SUBCORES = 16
NUM_WORKERS = NUM_CORES * NUM_SUBCORES
B_PER_W = BATCH // NUM_WORKERS
LANES = 16

_mesh = plsc.VectorSubcoreMesh(core_axis_name="c", subcore_axis_name="s")


@functools.partial(
    pl.kernel,
    mesh=_mesh,
    compiler_params=pltpu.CompilerParams(needs_layout_passes=False),
    out_type=jax.ShapeDtypeStruct((BATCH, 1, EMB_DIM), jnp.float32),
    scratch_types=[
        pltpu.VMEM((B_PER_W,), jnp.int32),
        pltpu.VMEM((B_PER_W, 1, EMB_DIM), jnp.float32),
        pltpu.SemaphoreType.DMA,
    ],
)
def _sc_gather(idx_hbm, table_hbm, out_hbm, idx_v, rows_v, sem):
    wid = lax.axis_index("s") * NUM_CORES + lax.axis_index("c")
    base = wid * B_PER_W
    lane_ids = lax.broadcasted_iota(jnp.int32, (LANES,), 0)

    pltpu.sync_copy(idx_hbm.at[pl.ds(base, B_PER_W)], idx_v)

    def group(g, _):
        v = idx_v[pl.ds(g * LANES, LANES)]
        for l in range(LANES):
            s = jnp.max(jnp.where(lane_ids == l, v, 0))
            pltpu.async_copy(
                table_hbm.at[pl.ds(s >> 3, 1), pl.ds(s & 7, 1)],
                rows_v.at[pl.ds(g * LANES + l, 1)],
                sem,
            )
        return ()

    lax.fori_loop(0, B_PER_W // LANES, group, (), unroll=4)

    pltpu.make_async_copy(
        table_hbm.at[pl.ds(0, B_PER_W), pl.ds(0, 1)], rows_v, sem).wait()

    pltpu.sync_copy(rows_v, out_hbm.at[pl.ds(base, B_PER_W)])


def kernel(x, table):
    idx = x.astype(jnp.int32)
    out3 = _sc_gather(idx, table.reshape(VOCAB // 8, 8, EMB_DIM))
    return out3.reshape(BATCH, EMB_DIM)


# speedup vs baseline: 2.5719x; 1.0041x over previous
import functools

import jax
import jax.numpy as jnp
from jax import lax
from jax.experimental import pallas as pl
from jax.experimental.pallas import tpu as pltpu
from jax.experimental.pallas import tpu_sc as plsc

VOCAB = 1000000
EMB_DIM = 64
BATCH = 16384

NUM_CORES = 2
NUM_SUBCORES = 16
NUM_WORKERS = NUM_CORES * NUM_SUBCORES
B_PER_W = BATCH // NUM_WORKERS
LANES = 16

_mesh = plsc.VectorSubcoreMesh(core_axis_name="c", subcore_axis_name="s")


@functools.partial(
    pl.kernel,
    mesh=_mesh,
    compiler_params=pltpu.CompilerParams(needs_layout_passes=False),
    out_type=jax.ShapeDtypeStruct((BATCH, 1, EMB_DIM), jnp.float32),
    scratch_types=[
        pltpu.VMEM((B_PER_W,), jnp.int32),
        pltpu.VMEM((B_PER_W, 1, EMB_DIM), jnp.float32),
        pltpu.SemaphoreType.DMA,
    ],
)
def _sc_gather(idx_hbm, table_hbm, out_hbm, idx_v, rows_v, sem):
    wid = lax.axis_index("s") * NUM_CORES + lax.axis_index("c")
    base = wid * B_PER_W
    lane_ids = lax.broadcasted_iota(jnp.int32, (LANES,), 0)

    pltpu.sync_copy(idx_hbm.at[pl.ds(base, B_PER_W)], idx_v)

    def group(g, _):
        v = idx_v[pl.ds(g * LANES, LANES)]
        for l in range(LANES):
            s = jnp.max(jnp.where(lane_ids == l, v, 0))
            pltpu.async_copy(
                table_hbm.at[pl.ds(s >> 3, 1), pl.ds(s & 7, 1)],
                rows_v.at[pl.ds(g * LANES + l, 1)],
                sem,
            )
        return ()

    lax.fori_loop(0, B_PER_W // LANES, group, (), unroll=False)

    pltpu.make_async_copy(
        table_hbm.at[pl.ds(0, B_PER_W), pl.ds(0, 1)], rows_v, sem).wait()

    pltpu.sync_copy(rows_v, out_hbm.at[pl.ds(base, B_PER_W)])


def kernel(x, table):
    idx = x.astype(jnp.int32)
    out3 = _sc_gather(idx, table.reshape(VOCAB // 8, 8, EMB_DIM))
    return out3.reshape(BATCH, EMB_DIM)
